# baseline (speedup 1.0000x reference)
import jax
import jax.numpy as jnp
from jax import lax
from jax.experimental import pallas as pl
from jax.experimental.pallas import tpu as pltpu
from jax.experimental.pallas import tpu_sc as plsc

B = 4096
L = 200
LP = 208
D = 32
NW = 32
RPW = B // NW
C0 = 104
C1 = L - C0


def _body(text_hbm, len_hbm, idf_hbm, table_hbm, out_hbm,
          text_v, len_v, idx0, idx1, rows0, rows1, idfs0, idfs1, out_v,
          sem0, sem1):
    wid = lax.axis_index("s") * 2 + lax.axis_index("c")
    base = wid * RPW
    bufs = ((idx0, rows0, idfs0, sem0), (idx1, rows1, idfs1, sem1))

    pltpu.sync_copy(text_hbm.at[pl.ds(base, RPW)], text_v)
    pltpu.sync_copy(len_hbm.at[pl.ds(base, RPW)], len_v)

    z = jnp.zeros((16,), jnp.float32)
    for _, rv, iv, _ in bufs:
        iv[pl.ds(LP - 16, 16)] = z
        for lpad in range(L, LP):
            rv[lpad, pl.ds(0, 16)] = z
            rv[lpad, pl.ds(16, 16)] = z

    def fire(r, slot):
        ix, rv, iv, sem = bufs[slot]
        i0 = text_v.at[r, pl.ds(0, C0)]
        i1 = text_v.at[r, pl.ds(C0, C1)]
        pltpu.async_copy(table_hbm.at[i0], rv.at[pl.ds(0, C0)], sem)
        pltpu.async_copy(table_hbm.at[i1], rv.at[pl.ds(C0, C1)], sem)
        pltpu.async_copy(idf_hbm.at[i0], iv.at[pl.ds(0, C0)], sem)
        pltpu.async_copy(idf_hbm.at[i1], iv.at[pl.ds(C0, C1)], sem)

    def wait_slot(slot):
        _, rv, iv, sem = bufs[slot]
        pltpu.make_async_copy(table_hbm.at[pl.ds(0, C0)], rv.at[pl.ds(0, C0)], sem).wait()
        pltpu.make_async_copy(table_hbm.at[pl.ds(0, C1)], rv.at[pl.ds(C0, C1)], sem).wait()
        pltpu.make_async_copy(idf_hbm.at[pl.ds(0, C0)], iv.at[pl.ds(0, C0)], sem).wait()
        pltpu.make_async_copy(idf_hbm.at[pl.ds(0, C1)], iv.at[pl.ds(C0, C1)], sem).wait()

    fire(0, 0)

    def do_block(rb, carry):
        inv = 1.0 / len_v[pl.ds(rb * 16, 16)].astype(jnp.float32)
        for j in range(16):
            r = rb * 16 + j
            slot = j % 2
            _, rv, iv, _ = bufs[slot]

            @pl.when(r + 1 < RPW)
            def _():
                fire(r + 1, (j + 1) % 2)

            wait_slot(slot)

            def acc_step(lb, accs):
                a0, a1 = accs
                w = iv[pl.ds(lb * 16, 16)]
                for jj in range(16):
                    l = lb * 16 + jj
                    s = w[jj]
                    a0 = a0 + rv[l, pl.ds(0, 16)] * s
                    a1 = a1 + rv[l, pl.ds(16, 16)] * s
                return (a0, a1)

            a0, a1 = lax.fori_loop(0, LP // 16, acc_step, (z, z))
            siv = inv[j]
            out_v[r, pl.ds(0, 16)] = a0 * siv
            out_v[r, pl.ds(16, 16)] = a1 * siv
        return carry

    lax.fori_loop(0, RPW // 16, do_block, 0)
    pltpu.sync_copy(out_v, out_hbm.at[pl.ds(base, RPW)])


@jax.jit
def _run(text, text_len, idf_flat, table):
    mesh = plsc.VectorSubcoreMesh(core_axis_name="c", subcore_axis_name="s")
    f = pl.kernel(
        _body,
        out_type=jax.ShapeDtypeStruct((B, D), jnp.float32),
        mesh=mesh,
        compiler_params=pltpu.CompilerParams(use_tc_tiling_on_sc=False),
        scratch_types=[
            pltpu.VMEM((RPW, L), jnp.int32),
            pltpu.VMEM((RPW,), jnp.int32),
            pltpu.VMEM((L,), jnp.int32),
            pltpu.VMEM((L,), jnp.int32),
            pltpu.VMEM((LP, D), jnp.float32),
            pltpu.VMEM((LP, D), jnp.float32),
            pltpu.VMEM((LP,), jnp.float32),
            pltpu.VMEM((LP,), jnp.float32),
            pltpu.VMEM((RPW, D), jnp.float32),
            pltpu.SemaphoreType.DMA,
            pltpu.SemaphoreType.DMA,
        ],
    )
    return f(text, text_len, idf_flat, table)


def kernel(text, text_len, idf, table):
    return _run(text, text_len, idf.reshape(-1), table)


# speedup vs baseline: 1.0210x; 1.0210x over previous
import jax
import jax.numpy as jnp
from jax import lax
from jax.experimental import pallas as pl
from jax.experimental.pallas import tpu as pltpu
from jax.experimental.pallas import tpu_sc as plsc

B = 4096
L = 200
D = 32
NW = 32
RPW = B // NW
LB = 8
NC = L // LB


def _body(text_hbm, len_hbm, idf_hbm, table_hbm, out_hbm,
          text_v, len_v, rows0, rows1, idfw0, idfw1, acc_v, sem0, sem1):
    wid = lax.axis_index("s") * 2 + lax.axis_index("c")
    base = wid * RPW
    bufs = ((rows0, idfw0, sem0), (rows1, idfw1, sem1))

    for I in range(NC):
        pltpu.sync_copy(text_hbm.at[I, wid], text_v.at[pl.ds(I * LB, LB)])
    pltpu.sync_copy(len_hbm.at[pl.ds(base, RPW)], len_v)

    z = jnp.zeros((16,), jnp.float32)

    def zero_blk(jb, carry):
        for jj in range(16):
            acc_v[jb * 16 + jj, pl.ds(0, 16)] = z
            acc_v[jb * 16 + jj, pl.ds(16, 16)] = z
        return carry

    lax.fori_loop(0, RPW // 16, zero_blk, 0)

    def fire(lb, slot):
        rv, wv, sem = bufs[slot]
        for li in range(LB):
            ix = text_v.at[lb * LB + li]
            pltpu.async_copy(table_hbm.at[ix], rv.at[li], sem)
            pltpu.async_copy(idf_hbm.at[ix], wv.at[li], sem)

    def wait_slot(slot):
        rv, wv, sem = bufs[slot]
        for li in range(LB):
            pltpu.make_async_copy(table_hbm.at[pl.ds(0, RPW)], rv.at[li], sem).wait()
            pltpu.make_async_copy(idf_hbm.at[pl.ds(0, RPW)], wv.at[li], sem).wait()

    fire(0, 0)

    def do_chunk(lb, carry):
        slot = lax.rem(lb, 2)
        for s in range(2):

            @pl.when(slot == s)
            def _():
                rv, wv, _ = bufs[s]

                @pl.when(lb + 1 < NC)
                def _():
                    fire(lb + 1, 1 - s)

                wait_slot(s)

                def do_jblk(jb, c2):
                    ws = [wv[li, pl.ds(jb * 16, 16)] for li in range(LB)]
                    for jj in range(16):
                        j = jb * 16 + jj
                        a0 = acc_v[j, pl.ds(0, 16)]
                        a1 = acc_v[j, pl.ds(16, 16)]
                        for li in range(LB):
                            s_w = ws[li][jj]
                            a0 = a0 + rv[li, j, pl.ds(0, 16)] * s_w
                            a1 = a1 + rv[li, j, pl.ds(16, 16)] * s_w
                        acc_v[j, pl.ds(0, 16)] = a0
                        acc_v[j, pl.ds(16, 16)] = a1
                    return c2

                lax.fori_loop(0, RPW // 16, do_jblk, 0)

        return carry

    lax.fori_loop(0, NC, do_chunk, 0)

    def norm_blk(jb, carry):
        inv = 1.0 / len_v[pl.ds(jb * 16, 16)].astype(jnp.float32)
        for jj in range(16):
            j = jb * 16 + jj
            siv = inv[jj]
            acc_v[j, pl.ds(0, 16)] = acc_v[j, pl.ds(0, 16)] * siv
            acc_v[j, pl.ds(16, 16)] = acc_v[j, pl.ds(16, 16)] * siv
        return carry

    lax.fori_loop(0, RPW // 16, norm_blk, 0)
    pltpu.sync_copy(acc_v, out_hbm.at[pl.ds(base, RPW)])


@jax.jit
def _run(text_p, text_len, idf_flat, table):
    mesh = plsc.VectorSubcoreMesh(core_axis_name="c", subcore_axis_name="s")
    f = pl.kernel(
        _body,
        out_type=jax.ShapeDtypeStruct((B, D), jnp.float32),
        mesh=mesh,
        compiler_params=pltpu.CompilerParams(use_tc_tiling_on_sc=False),
        scratch_types=[
            pltpu.VMEM((L, RPW), jnp.int32),
            pltpu.VMEM((RPW,), jnp.int32),
            pltpu.VMEM((LB, RPW, D), jnp.float32),
            pltpu.VMEM((LB, RPW, D), jnp.float32),
            pltpu.VMEM((LB, RPW), jnp.float32),
            pltpu.VMEM((LB, RPW), jnp.float32),
            pltpu.VMEM((RPW, D), jnp.float32),
            pltpu.SemaphoreType.DMA,
            pltpu.SemaphoreType.DMA,
        ],
    )
    return f(text_p, text_len, idf_flat, table)


def kernel(text, text_len, idf, table):
    text_p = text.T.reshape(L // 8, 8, B // 128, 128).transpose(0, 2, 1, 3)
    return _run(text_p, text_len, idf.reshape(-1), table)
